# baseline (speedup 1.0000x reference)
import jax
import jax.numpy as jnp
from jax import lax
from jax.experimental import pallas as pl
from jax.experimental.pallas import tpu as pltpu
from jax.experimental.pallas import tpu_sc as plsc

_BATCH = 16384
_DIM = 128
_NC = 2
_NS = 16
_NW = _NC * _NS
_BPW = _BATCH // _NW
_P = 64
_NBUF = 6
_LEAD = 5


def _sc_gather_body(u_hbm, ui_hbm, i_hbm, ii_hbm, ou_hbm, oi_hbm,
                    uidx_v, iidx_v, bufs_and_sems):
    bufs = bufs_and_sems[:_NBUF]
    gsem = bufs_and_sems[_NBUF:2 * _NBUF]
    wsem = bufs_and_sems[2 * _NBUF:]
    wid = lax.axis_index("s") * _NC + lax.axis_index("c")
    base = wid * _BPW
    uic = pltpu.async_copy(ui_hbm.at[pl.ds(base, _BPW)], uidx_v, gsem[0])
    iic = pltpu.async_copy(ii_hbm.at[pl.ds(base, _BPW)], iidx_v, gsem[1])
    uic.wait()
    iic.wait()

    npieces = _BPW // _P
    items = []
    for p in range(npieces):
        items.append((u_hbm, uidx_v, ou_hbm, p))
        items.append((i_hbm, iidx_v, oi_hbm, p))
    n = len(items)

    def start_gather(j):
        tab, idx, _, p = items[j]
        b = j % _NBUF
        return pltpu.async_copy(
            tab.at[idx.at[pl.ds(p * _P, _P)]], bufs[b], gsem[b])

    gcp = {}
    wcp = {}
    for j in range(min(_LEAD, n)):
        gcp[j] = start_gather(j)
    for j in range(n):
        b = j % _NBUF
        gcp[j].wait()
        _, _, out, p = items[j]
        wcp[j] = pltpu.async_copy(
            bufs[b], out.at[pl.ds(base + p * _P, _P)], wsem[b])
        nxt = j + _LEAD
        if nxt < n:
            prev = nxt - _NBUF
            if prev >= 0:
                wcp[prev].wait()
            gcp[nxt] = start_gather(nxt)
    for j in range(max(0, n - _NBUF), n):
        wcp[j].wait()


def _sc_gather(uEmbed, userIdx, iEmbed, itemIdx):
    mesh = plsc.VectorSubcoreMesh(core_axis_name="c", subcore_axis_name="s")
    scratch = (
        [pltpu.VMEM((_BPW,), jnp.int32), pltpu.VMEM((_BPW,), jnp.int32)]
        + [pltpu.VMEM((_P, _DIM), jnp.float32) for _ in range(_NBUF)]
        + [pltpu.SemaphoreType.DMA for _ in range(2 * _NBUF)]
    )

    def body(u_hbm, ui_hbm, i_hbm, ii_hbm, ou_hbm, oi_hbm, uidx_v, iidx_v,
             *bufs_and_sems):
        _sc_gather_body(u_hbm, ui_hbm, i_hbm, ii_hbm, ou_hbm, oi_hbm,
                        uidx_v, iidx_v, bufs_and_sems)

    k = pl.kernel(
        body,
        mesh=mesh,
        out_type=(
            jax.ShapeDtypeStruct((_BATCH, _DIM), jnp.float32),
            jax.ShapeDtypeStruct((_BATCH, _DIM), jnp.float32),
        ),
        scratch_types=scratch,
    )
    return k(uEmbed, userIdx, iEmbed, itemIdx)


_HID = 64
_BB = 8192


def _mlp_body(u_ref, i_ref, w_ref, b1_ref, w2_ref, b2_ref, o_ref):
    u = u_ref[...].astype(jnp.bfloat16)
    i = i_ref[...].astype(jnp.bfloat16)
    w = w_ref[...].astype(jnp.bfloat16)
    dn = (((1,), (1,)), ((), ()))
    h = jax.lax.dot_general(u, w[:, :_DIM], dn,
                            preferred_element_type=jnp.float32)
    h = h + jax.lax.dot_general(i, w[:, _DIM:], dn,
                                preferred_element_type=jnp.float32)
    h = jnp.maximum(h + b1_ref[...], 0.0)
    z = jnp.sum(h * w2_ref[...], axis=1)
    o_ref[...] = jax.nn.sigmoid(z + b2_ref[...])


def _mlp(uG, iG, W_cvr, b_cvr, W_cvr1, b_cvr1):
    return pl.pallas_call(
        _mlp_body,
        grid=(_BATCH // _BB,),
        in_specs=[
            pl.BlockSpec((_BB, _DIM), lambda j: (j, 0)),
            pl.BlockSpec((_BB, _DIM), lambda j: (j, 0)),
            pl.BlockSpec((_HID, 2 * _DIM), lambda j: (0, 0)),
            pl.BlockSpec((1, _HID), lambda j: (0, 0)),
            pl.BlockSpec((1, _HID), lambda j: (0, 0)),
            pl.BlockSpec((1,), lambda j: (0,)),
        ],
        out_specs=pl.BlockSpec((_BB,), lambda j: (j,)),
        out_shape=jax.ShapeDtypeStruct((_BATCH,), jnp.float32),
    )(uG, iG, W_cvr, b_cvr.reshape(1, _HID), W_cvr1, b_cvr1)


def kernel(userIdx, itemIdx, uEmbed, iEmbed, W_cvr, b_cvr, W_cvr1, b_cvr1):
    userIdx = userIdx.astype(jnp.int32)
    itemIdx = itemIdx.astype(jnp.int32)
    uG, iG = _sc_gather(uEmbed, userIdx, iEmbed, itemIdx)
    return _mlp(uG, iG, W_cvr, b_cvr, W_cvr1, b_cvr1)


# speedup vs baseline: 1.0196x; 1.0196x over previous
import jax
import jax.numpy as jnp
from jax import lax
from jax.experimental import pallas as pl
from jax.experimental.pallas import tpu as pltpu
from jax.experimental.pallas import tpu_sc as plsc

_BATCH = 16384
_DIM = 128
_NC = 2
_NS = 16
_NW = _NC * _NS
_BPW = _BATCH // _NW
_P = 256
_NBUF = 3
_LEAD = 2


def _sc_gather_body(u_hbm, ui_hbm, i_hbm, ii_hbm, ou_hbm, oi_hbm,
                    uidx_v, iidx_v, bufs_and_sems):
    bufs = bufs_and_sems[:_NBUF]
    gsem = bufs_and_sems[_NBUF:2 * _NBUF]
    wsem = bufs_and_sems[2 * _NBUF:]
    wid = lax.axis_index("s") * _NC + lax.axis_index("c")
    base = wid * _BPW
    uic = pltpu.async_copy(ui_hbm.at[pl.ds(base, _BPW)], uidx_v, gsem[0])
    iic = pltpu.async_copy(ii_hbm.at[pl.ds(base, _BPW)], iidx_v, gsem[1])
    uic.wait()
    iic.wait()

    npieces = _BPW // _P
    items = []
    for p in range(npieces):
        items.append((u_hbm, uidx_v, ou_hbm, p))
        items.append((i_hbm, iidx_v, oi_hbm, p))
    n = len(items)

    def start_gather(j):
        tab, idx, _, p = items[j]
        b = j % _NBUF
        return pltpu.async_copy(
            tab.at[idx.at[pl.ds(p * _P, _P)]], bufs[b], gsem[b])

    gcp = {}
    wcp = {}
    for j in range(min(_LEAD, n)):
        gcp[j] = start_gather(j)
    for j in range(n):
        b = j % _NBUF
        gcp[j].wait()
        _, _, out, p = items[j]
        wcp[j] = pltpu.async_copy(
            bufs[b], out.at[pl.ds(base + p * _P, _P)], wsem[b])
        nxt = j + _LEAD
        if nxt < n:
            prev = nxt - _NBUF
            if prev >= 0:
                wcp[prev].wait()
            gcp[nxt] = start_gather(nxt)
    for j in range(max(0, n - _NBUF), n):
        wcp[j].wait()


def _sc_gather(uEmbed, userIdx, iEmbed, itemIdx):
    mesh = plsc.VectorSubcoreMesh(core_axis_name="c", subcore_axis_name="s")
    scratch = (
        [pltpu.VMEM((_BPW,), jnp.int32), pltpu.VMEM((_BPW,), jnp.int32)]
        + [pltpu.VMEM((_P, _DIM), jnp.float32) for _ in range(_NBUF)]
        + [pltpu.SemaphoreType.DMA for _ in range(2 * _NBUF)]
    )

    def body(u_hbm, ui_hbm, i_hbm, ii_hbm, ou_hbm, oi_hbm, uidx_v, iidx_v,
             *bufs_and_sems):
        _sc_gather_body(u_hbm, ui_hbm, i_hbm, ii_hbm, ou_hbm, oi_hbm,
                        uidx_v, iidx_v, bufs_and_sems)

    k = pl.kernel(
        body,
        mesh=mesh,
        out_type=(
            jax.ShapeDtypeStruct((_BATCH, _DIM), jnp.float32),
            jax.ShapeDtypeStruct((_BATCH, _DIM), jnp.float32),
        ),
        scratch_types=scratch,
    )
    return k(uEmbed, userIdx, iEmbed, itemIdx)


_HID = 64
_BB = 8192


def _mlp_body(u_ref, i_ref, w_ref, b1_ref, w2_ref, b2_ref, o_ref):
    u = u_ref[...].astype(jnp.bfloat16)
    i = i_ref[...].astype(jnp.bfloat16)
    w = w_ref[...].astype(jnp.bfloat16)
    dn = (((1,), (1,)), ((), ()))
    h = jax.lax.dot_general(u, w[:, :_DIM], dn,
                            preferred_element_type=jnp.float32)
    h = h + jax.lax.dot_general(i, w[:, _DIM:], dn,
                                preferred_element_type=jnp.float32)
    h = jnp.maximum(h + b1_ref[...], 0.0)
    z = jnp.sum(h * w2_ref[...], axis=1)
    o_ref[...] = jax.nn.sigmoid(z + b2_ref[...])


def _mlp(uG, iG, W_cvr, b_cvr, W_cvr1, b_cvr1):
    return pl.pallas_call(
        _mlp_body,
        grid=(_BATCH // _BB,),
        in_specs=[
            pl.BlockSpec((_BB, _DIM), lambda j: (j, 0)),
            pl.BlockSpec((_BB, _DIM), lambda j: (j, 0)),
            pl.BlockSpec((_HID, 2 * _DIM), lambda j: (0, 0)),
            pl.BlockSpec((1, _HID), lambda j: (0, 0)),
            pl.BlockSpec((1, _HID), lambda j: (0, 0)),
            pl.BlockSpec((1,), lambda j: (0,)),
        ],
        out_specs=pl.BlockSpec((_BB,), lambda j: (j,)),
        out_shape=jax.ShapeDtypeStruct((_BATCH,), jnp.float32),
    )(uG, iG, W_cvr, b_cvr.reshape(1, _HID), W_cvr1, b_cvr1)


def kernel(userIdx, itemIdx, uEmbed, iEmbed, W_cvr, b_cvr, W_cvr1, b_cvr1):
    userIdx = userIdx.astype(jnp.int32)
    itemIdx = itemIdx.astype(jnp.int32)
    uG, iG = _sc_gather(uEmbed, userIdx, iEmbed, itemIdx)
    return _mlp(uG, iG, W_cvr, b_cvr, W_cvr1, b_cvr1)
